# baseline (speedup 1.0000x reference)
import functools
import math

import jax
import jax.numpy as jnp
from jax import lax
from jax.experimental import pallas as pl
from jax.experimental.pallas import tpu as pltpu
from jax.experimental.pallas import tpu_sc as plsc

MEM = 2048
CODE = 128
NB = 8
NW = 16
RPW = MEM // NW
NEG_BIG = -1e30
DN = (((1,), (1,)), ((), ()))


def _proj_prep_kernel(x_ref, wproj_ref, bproj_ref, mask_ref, wq_ref, bq_ref,
                      mem_ref, f_ref, q_ref, pooled_ref, pooledn_ref,
                      memn_ref, memnt_blk_ref, lt_ref, pooled_sc):
    g = pl.program_id(0)
    hw = x_ref.shape[-1]
    x = x_ref[0]
    f = jnp.dot(wproj_ref[...], x, preferred_element_type=jnp.float32)
    f = f + bproj_ref[...]
    f_ref[0] = f
    mask = mask_ref[0]
    pooled = jnp.dot(f, mask.T, preferred_element_type=jnp.float32) / hw
    pooled_sc[pl.ds(g, 1), :] = pooled.T
    ql = jnp.dot(wq_ref[...], f, preferred_element_type=jnp.float32)
    ql = ql + bq_ref[0, 0]
    qm = jnp.max(ql, axis=-1, keepdims=True)
    qe = jnp.exp(ql - qm)
    q_ref[0] = qe / jnp.sum(qe, axis=-1, keepdims=True)

    @pl.when(g == NB - 1)
    def prep():
        mem = mem_ref[...]
        inv = lax.rsqrt(jnp.sum(mem * mem, axis=1, keepdims=True))
        memn = mem * inv
        memn_ref[...] = memn
        memnt = memn.T
        for w in range(NW):
            memnt_blk_ref[w] = memnt[:, w * RPW:(w + 1) * RPW]
        pooled_all = pooled_sc[...]
        pooled_ref[...] = pooled_all
        pinv = lax.rsqrt(jnp.sum(pooled_all * pooled_all, axis=1,
                                 keepdims=True))
        pooledn = pooled_all * pinv
        pooledn_ref[...] = pooledn
        lt_ref[...] = jnp.dot(pooledn, memnt,
                              preferred_element_type=jnp.float32)


GDN = lax.GatherDimensionNumbers(offset_dims=(), collapsed_slice_dims=(0,),
                                 start_index_map=(0,))


def _shuf(v, idx16):
    return lax.gather(v, idx16[:, None], GDN, slice_sizes=(1,),
                      mode=lax.GatherScatterMode.PROMISE_IN_BOUNDS)


def _lane():
    return lax.iota(jnp.int32, 16)


def _allreduce(v, op):
    lane = _lane()
    for sh in (8, 4, 2, 1):
        v = op(v, _shuf(v, (lane + sh) & 15))
    return v


def _sc_update_body(mem_hbm, memn_hbm, memnt_blk_hbm, pooled_hbm,
                    pooledn_hbm, lt_hbm, out_hbm,
                    slab_v, slabn_v, slabt_v, lt_v, pooled_v, pooledn_v,
                    row_v, nrn_v, candv_my, candi_my, candv_v, candi_v,
                    row_sh, nrn_sh, candv_sh, candi_sh):
    cid = lax.axis_index("c")
    sid = lax.axis_index("s")
    lane = _lane()
    ngrp = RPW // 16
    nch = CODE // 16

    @pl.when(cid == 0)
    def active():
        base = sid * RPW
        pltpu.sync_copy(mem_hbm.at[pl.ds(base * CODE, RPW * CODE)], slab_v)
        pltpu.sync_copy(memn_hbm.at[pl.ds(base * CODE, RPW * CODE)], slabn_v)
        pltpu.sync_copy(memnt_blk_hbm.at[sid], slabt_v)
        pltpu.sync_copy(pooled_hbm, pooled_v)
        pltpu.sync_copy(pooledn_hbm, pooledn_v)
        pltpu.sync_copy(lt_hbm, lt_v)

        def step(i, _):
            def chunk(k, carry):
                mx, ck = carry
                v = lt_v[pl.ds(i * MEM + k * 16, 16)]
                better = v > mx
                mx = jnp.where(better, v, mx)
                ck = jnp.where(better, jnp.full((16,), k, jnp.int32), ck)
                return mx, ck
            mx, ck = lax.fori_loop(0, MEM // 16, chunk,
                                   (jnp.full((16,), NEG_BIG, jnp.float32),
                                    jnp.zeros((16,), jnp.int32)))
            value_i = _allreduce(mx, jnp.maximum)
            cols = jnp.where(mx == value_i, ck * 16 + lane, MEM)
            index_v = _allreduce(cols, jnp.minimum)
            idx_s = index_v[0]
            owner_s = idx_s // RPW
            loc_s = idx_s - owner_s * RPW

            @pl.when(sid == owner_s)
            def pub_row():
                pltpu.sync_copy(slabn_v.at[pl.ds(loc_s * CODE, CODE)], row_sh)
            plsc.subcore_barrier()
            pltpu.sync_copy(row_sh, row_v)

            accs = [jnp.zeros((16,), jnp.float32) for _ in range(ngrp)]
            for j in range(nch):
                rv = row_v[pl.ds(j * 16, 16)]
                for l in range(16):
                    sc = jnp.full((16,), rv[l], jnp.float32)
                    d = j * 16 + l
                    for gi in range(ngrp):
                        accs[gi] = accs[gi] + \
                            slabt_v[pl.ds(d * RPW + gi * 16, 16)] * sc
            lmaxv = jnp.full((16,), NEG_BIG, jnp.float32)
            lrowv = jnp.full((16,), MEM, jnp.int32)
            for gi in range(ngrp):
                rows = base + gi * 16 + lane
                vals = jnp.where(rows == index_v, NEG_BIG, accs[gi])
                better = vals > lmaxv
                lrowv = jnp.where(better, rows, lrowv)
                lmaxv = jnp.where(better, vals, lmaxv)
            lmax = _allreduce(lmaxv, jnp.maximum)
            lrow = _allreduce(jnp.where(lmaxv == lmax, lrowv, MEM),
                              jnp.minimum)
            candv_my[...] = lmax
            candi_my[...] = lrow
            pltpu.sync_copy(candv_my, candv_sh.at[pl.ds(sid * 16, 16)])
            pltpu.sync_copy(candi_my, candi_sh.at[pl.ds(sid * 16, 16)])
            plsc.subcore_barrier()

            pltpu.sync_copy(candv_sh, candv_v)
            pltpu.sync_copy(candi_sh, candi_v)
            gmx = jnp.full((16,), NEG_BIG, jnp.float32)
            gix = jnp.full((16,), MEM, jnp.int32)
            for w in range(NW):
                vw = candv_v[pl.ds(w * 16, 16)]
                iw = candi_v[pl.ds(w * 16, 16)]
                better = vw > gmx
                tie = (vw == gmx) & (iw < gix)
                gix = jnp.where(better | tie, iw, gix)
                gmx = jnp.where(better, vw, gmx)
            hard_s = gix[0]
            vq_chunk = lt_v[pl.ds(i * MEM + (hard_s // 16) * 16, 16)]
            value_q = _shuf(vq_chunk, jnp.full((16,), hard_s % 16, jnp.int32))
            rate = value_q / (value_q + value_i)

            @pl.when(sid == owner_s)
            def upd():
                acc = jnp.zeros((16,), jnp.float32)
                for j in range(nch):
                    off = loc_s * CODE + j * 16
                    nr = (slab_v[pl.ds(off, 16)] * rate +
                          (1.0 - rate) * pooled_v[pl.ds(i * CODE + j * 16, 16)])
                    slab_v[pl.ds(off, 16)] = nr
                    acc = acc + nr * nr
                s = _allreduce(acc, jnp.add)
                t = 0.5 * (1.0 + s)
                for _n in range(20):
                    t = 0.5 * (t + s / t)
                y = 1.0 / t
                colbase = (loc_s // 16) * 16
                lpos = jnp.full((16,), loc_s % 16, jnp.int32)
                for j in range(nch):
                    off = loc_s * CODE + j * 16
                    nrn = slab_v[pl.ds(off, 16)] * y
                    slabn_v[pl.ds(off, 16)] = nrn
                    nrn_v[pl.ds(j * 16, 16)] = nrn
                    for l in range(16):
                        d = j * 16 + l
                        w0 = d * RPW + colbase
                        ch = slabt_v[pl.ds(w0, 16)]
                        slabt_v[pl.ds(w0, 16)] = jnp.where(
                            lane == lpos, jnp.full((16,), nrn[l]), ch)
                pltpu.sync_copy(nrn_v, nrn_sh)
            plsc.subcore_barrier()
            pltpu.sync_copy(nrn_sh, nrn_v)

            cb2 = (idx_s // 16) * 16
            lpos2 = jnp.full((16,), idx_s % 16, jnp.int32)
            for ii in range(NB):
                acc = jnp.zeros((16,), jnp.float32)
                for j in range(nch):
                    acc = acc + (pooledn_v[pl.ds(ii * CODE + j * 16, 16)] *
                                 nrn_v[pl.ds(j * 16, 16)])
                corr = _allreduce(acc, jnp.add)
                w0 = ii * MEM + cb2
                ch = lt_v[pl.ds(w0, 16)]
                lt_v[pl.ds(w0, 16)] = jnp.where(lane == lpos2, corr, ch)
            return 0

        lax.fori_loop(0, NB, step, 0)
        pltpu.sync_copy(slab_v, out_hbm.at[pl.ds(base * CODE, RPW * CODE)])


def _sc_update(memory, memn, memnt_blk, pooled, pooledn, lt0):
    mesh = plsc.VectorSubcoreMesh(core_axis_name="c", subcore_axis_name="s")
    kern = functools.partial(
        pl.kernel, mesh=mesh,
        out_type=jax.ShapeDtypeStruct((MEM * CODE,), jnp.float32),
        scratch_types=[
            pltpu.VMEM((RPW * CODE,), jnp.float32),
            pltpu.VMEM((RPW * CODE,), jnp.float32),
            pltpu.VMEM((RPW * CODE,), jnp.float32),
            pltpu.VMEM((NB * MEM,), jnp.float32),
            pltpu.VMEM((NB * CODE,), jnp.float32),
            pltpu.VMEM((NB * CODE,), jnp.float32),
            pltpu.VMEM((CODE,), jnp.float32),
            pltpu.VMEM((CODE,), jnp.float32),
            pltpu.VMEM((16,), jnp.float32),
            pltpu.VMEM((16,), jnp.int32),
            pltpu.VMEM((NW * 16,), jnp.float32),
            pltpu.VMEM((NW * 16,), jnp.int32),
            pltpu.VMEM_SHARED((CODE,), jnp.float32),
            pltpu.VMEM_SHARED((CODE,), jnp.float32),
            pltpu.VMEM_SHARED((NW * 16,), jnp.float32),
            pltpu.VMEM_SHARED((NW * 16,), jnp.int32),
        ],
    )(_sc_update_body)
    return kern(memory.reshape(-1), memn.reshape(-1), memnt_blk,
                pooled.reshape(-1), pooledn.reshape(-1), lt0.reshape(-1))


def _attn_out_kernel(f_ref, mem_ref, q_ref, wv_ref, bv_ref, wz_ref, bz_ref,
                     lng_ref, lnb_ref, out_ref):
    fb = f_ref[0]
    mem = mem_ref[...]
    scale = 1.0 / math.sqrt(float(CODE))
    scores = jnp.dot(mem.astype(jnp.bfloat16),
                     (fb * scale).astype(jnp.bfloat16),
                     preferred_element_type=jnp.float32)
    e = jnp.exp(scores)
    denom = jnp.dot(jnp.ones((1, MEM), jnp.float32), e,
                    preferred_element_type=jnp.float32)
    w = q_ref[0] / denom
    a_bar = lax.dot_general(e, w, (((1,), (1,)), ((), ())),
                            preferred_element_type=jnp.float32)
    memv = lax.dot_general(mem, wv_ref[...], DN,
                           preferred_element_type=jnp.float32)
    z = jnp.dot(a_bar.T, memv, preferred_element_type=jnp.float32)
    z = z + bv_ref[...]
    z = lax.dot_general(z, wz_ref[...], DN,
                        preferred_element_type=jnp.float32) + bz_ref[...]
    mu = jnp.mean(z, axis=-1, keepdims=True)
    var = jnp.mean((z - mu) * (z - mu), axis=-1, keepdims=True)
    z = (z - mu) * lax.rsqrt(var + 1e-5) * lng_ref[...] + lnb_ref[...]
    gate = jax.nn.sigmoid(z)
    out_ref[0, :CODE, :] = fb
    out_ref[0, CODE:, :] = fb * gate.T


def kernel(feats, preds, memory, W_proj, b_proj, Wq, bq, Wv, bv, Wz, bz, ln_g, ln_b):
    B, Cin, H, W = feats.shape
    HW = H * W
    C = W_proj.shape[0]
    M = memory.shape[0]
    x = feats.reshape(B, Cin, HW)
    mask = preds.reshape(B, 1, HW)

    f, q, pooled, pooledn, memn, memnt_blk, lt0 = pl.pallas_call(
        _proj_prep_kernel,
        grid=(B,),
        in_specs=[
            pl.BlockSpec((1, Cin, HW), lambda g: (g, 0, 0)),
            pl.BlockSpec((C, Cin), lambda g: (0, 0)),
            pl.BlockSpec((C, 1), lambda g: (0, 0)),
            pl.BlockSpec((1, 1, HW), lambda g: (g, 0, 0)),
            pl.BlockSpec((1, C), lambda g: (0, 0)),
            pl.BlockSpec((1, 1), lambda g: (0, 0)),
            pl.BlockSpec((M, C), lambda g: (0, 0)),
        ],
        out_specs=[
            pl.BlockSpec((1, C, HW), lambda g: (g, 0, 0)),
            pl.BlockSpec((1, 1, HW), lambda g: (g, 0, 0)),
            pl.BlockSpec((NB, C), lambda g: (0, 0)),
            pl.BlockSpec((NB, C), lambda g: (0, 0)),
            pl.BlockSpec((M, C), lambda g: (0, 0)),
            pl.BlockSpec((NW, C, M // NW), lambda g: (0, 0, 0)),
            pl.BlockSpec((NB, M), lambda g: (0, 0)),
        ],
        out_shape=[
            jax.ShapeDtypeStruct((B, C, HW), jnp.float32),
            jax.ShapeDtypeStruct((B, 1, HW), jnp.float32),
            jax.ShapeDtypeStruct((NB, C), jnp.float32),
            jax.ShapeDtypeStruct((NB, C), jnp.float32),
            jax.ShapeDtypeStruct((M, C), jnp.float32),
            jax.ShapeDtypeStruct((NW, C, M // NW), jnp.float32),
            jax.ShapeDtypeStruct((NB, M), jnp.float32),
        ],
        scratch_shapes=[pltpu.VMEM((NB, C), jnp.float32)],
    )(x, W_proj, b_proj.reshape(C, 1), mask, Wq, bq.reshape(1, 1), memory)

    mem_new = _sc_update(memory, memn,
                         memnt_blk.reshape(NW, C * (M // NW)),
                         pooled, pooledn, lt0).reshape(M, C)

    out = pl.pallas_call(
        _attn_out_kernel,
        grid=(B,),
        in_specs=[
            pl.BlockSpec((1, C, HW), lambda g: (g, 0, 0)),
            pl.BlockSpec((M, C), lambda g: (0, 0)),
            pl.BlockSpec((1, 1, HW), lambda g: (g, 0, 0)),
            pl.BlockSpec((C // 2, C), lambda g: (0, 0)),
            pl.BlockSpec((1, C // 2), lambda g: (0, 0)),
            pl.BlockSpec((C, C // 2), lambda g: (0, 0)),
            pl.BlockSpec((1, C), lambda g: (0, 0)),
            pl.BlockSpec((1, C), lambda g: (0, 0)),
            pl.BlockSpec((1, C), lambda g: (0, 0)),
        ],
        out_specs=pl.BlockSpec((1, 2 * C, HW), lambda g: (g, 0, 0)),
        out_shape=jax.ShapeDtypeStruct((B, 2 * C, HW), jnp.float32),
    )(f, mem_new, q, Wv, bv.reshape(1, C // 2), Wz, bz.reshape(1, C),
      ln_g.reshape(1, C), ln_b.reshape(1, C))

    return out.reshape(B, 2 * C, H, W)
